# baseline (speedup 1.0000x reference)
import functools

import jax
import jax.numpy as jnp
from jax.experimental import pallas as pl
from jax.experimental.pallas import tpu as pltpu

NUM_EXPERTS = 8
TOP_K = 2
D_MODEL = 1024
D_FF = 2048
FBLK = 512
NF = D_FF // FBLK


def _router_body(x_ref, wr_ref, comb_ref):
    x = x_ref[...]
    wr = wr_ref[...]
    logits = jax.lax.dot_general(
        x, wr, (((1,), (1,)), ((), ())), preferred_element_type=jnp.float32)
    m = jnp.max(logits, axis=1, keepdims=True)
    ex = jnp.exp(logits - m)
    p = ex / jnp.sum(ex, axis=1, keepdims=True)
    lane = jax.lax.broadcasted_iota(jnp.int32, p.shape, 1)
    m0 = jnp.max(p, axis=1, keepdims=True)
    i0 = jnp.min(jnp.where(p == m0, lane, NUM_EXPERTS), axis=1, keepdims=True)
    p1 = jnp.where(lane == i0, -jnp.inf, p)
    m1 = jnp.max(p1, axis=1, keepdims=True)
    i1 = jnp.min(jnp.where(p1 == m1, lane, NUM_EXPERTS), axis=1, keepdims=True)
    s = m0 + m1
    comb_ref[...] = (jnp.where(lane == i0, m0 / s, 0.0)
                     + jnp.where(lane == i1, m1 / s, 0.0))


def _expert_body(comb_ref, x_ref, w1_ref, v1_ref, w2_ref, out_ref):
    e = pl.program_id(0)
    f = pl.program_id(1)
    x = x_ref[...].astype(jnp.bfloat16)
    w1 = w1_ref[0]
    v1 = v1_ref[0]
    w2 = w2_ref[0]
    a = jax.lax.dot_general(
        x, w1, (((1,), (1,)), ((), ())), preferred_element_type=jnp.float32)
    b = jax.lax.dot_general(
        x, v1, (((1,), (1,)), ((), ())), preferred_element_type=jnp.float32)
    h = ((a * jax.lax.logistic(a)) * b).astype(jnp.bfloat16)
    y = jax.lax.dot_general(
        h, w2, (((1,), (1,)), ((), ())), preferred_element_type=jnp.float32)
    ids = jax.lax.broadcasted_iota(jnp.int32, (NUM_EXPERTS, 1), 0)
    onehot = (ids == e).astype(jnp.float32)
    c = jax.lax.dot_general(
        comb_ref[...], onehot, (((1,), (0,)), ((), ())),
        preferred_element_type=jnp.float32)

    @pl.when(jnp.logical_and(e == 0, f == 0))
    def _():
        out_ref[...] = jnp.zeros_like(out_ref)

    out_ref[...] += y * c


@functools.partial(jax.jit, static_argnames=("interpret",))
def _run(x, w_router, ws, w2s, interpret=False):
    t = x.shape[0]
    comb = pl.pallas_call(
        _router_body,
        out_shape=jax.ShapeDtypeStruct((t, NUM_EXPERTS), jnp.float32),
        interpret=interpret,
    )(x, w_router)

    ws_b = ws.astype(jnp.bfloat16)
    w2s_b = w2s.astype(jnp.bfloat16)
    out = pl.pallas_call(
        _expert_body,
        grid=(NUM_EXPERTS, NF),
        in_specs=[
            pl.BlockSpec((t, NUM_EXPERTS), lambda e, f: (0, 0)),
            pl.BlockSpec((t, D_MODEL), lambda e, f: (0, 0)),
            pl.BlockSpec((1, FBLK, D_MODEL), lambda e, f: (e, f, 0)),
            pl.BlockSpec((1, FBLK, D_MODEL), lambda e, f: (e, NF + f, 0)),
            pl.BlockSpec((1, D_MODEL, FBLK), lambda e, f: (e, 0, f)),
        ],
        out_specs=pl.BlockSpec((t, D_MODEL), lambda e, f: (0, 0)),
        out_shape=jax.ShapeDtypeStruct((t, D_MODEL), jnp.float32),
        compiler_params=pltpu.CompilerParams(
            dimension_semantics=("arbitrary", "arbitrary")),
        interpret=interpret,
    )(comb, x, ws_b, ws_b, w2s_b)
    return out


def kernel(hidden_states, w_router, ws, w2s):
    b, s, d = hidden_states.shape
    x = hidden_states.reshape(s, d)
    out = _run(x, w_router, ws, w2s)
    return out.reshape(b, s, d)


# speedup vs baseline: 1.3146x; 1.3146x over previous
import functools

import jax
import jax.numpy as jnp
from jax import lax
from jax.experimental import pallas as pl
from jax.experimental.pallas import tpu as pltpu
from jax.experimental.pallas import tpu_sc as plsc

NUM_EXPERTS = 8
TOP_K = 2
D_MODEL = 1024
D_FF = 2048
T = 2048

BM = 128
NB = (T * TOP_K) // BM + NUM_EXPERTS
NPAD = NB * BM

NW = 32
TW = T // NW


def _route_body(x_ref, wr_ref, pos_ref, wn_ref, bmeta_ref):
    x = x_ref[...]
    wr = wr_ref[...]
    logits = lax.dot_general(
        x, wr, (((1,), (1,)), ((), ())), preferred_element_type=jnp.float32)
    m = jnp.max(logits, axis=1, keepdims=True)
    ex = jnp.exp(logits - m)
    p = ex / jnp.sum(ex, axis=1, keepdims=True)
    lane = lax.broadcasted_iota(jnp.int32, p.shape, 1)
    m0 = jnp.max(p, axis=1, keepdims=True)
    i0 = jnp.min(jnp.where(p == m0, lane, NUM_EXPERTS), axis=1, keepdims=True)
    p1m = jnp.where(lane == i0, -jnp.inf, p)
    m1 = jnp.max(p1m, axis=1, keepdims=True)
    i1 = jnp.min(jnp.where(p1m == m1, lane, NUM_EXPERTS), axis=1, keepdims=True)
    s = m0 + m1

    oh0 = (lane == i0).astype(jnp.int32)
    oh1 = (lane == i1).astype(jnp.int32)

    def cumsum_tok(a):
        sh = 1
        while sh < T:
            a = a + jnp.concatenate(
                [jnp.zeros((sh, NUM_EXPERTS), jnp.int32), a[:T - sh]], axis=0)
            sh *= 2
        return a

    c0 = cumsum_tok(oh0)
    c1 = cumsum_tok(oh1)
    tot0 = c0[T - 1:T, :]
    tot1 = c1[T - 1:T, :]
    cnt = tot0 + tot1
    cp = ((cnt + BM - 1) // BM) * BM

    ip = cp
    for sh in (1, 2, 4):
        ip = ip + jnp.concatenate(
            [jnp.zeros((1, sh), jnp.int32), ip[:, :NUM_EXPERTS - sh]], axis=1)
    off = ip - cp

    pos0 = jnp.sum(oh0 * (off + c0 - 1), axis=1, keepdims=True)
    pos1 = jnp.sum(oh1 * (off + tot0 + c1 - 1), axis=1, keepdims=True)
    pos_ref[:, 0:1] = pos0
    pos_ref[:, 1:2] = pos1
    wn_ref[:, 0:1] = m0 / s
    wn_ref[:, 1:2] = m1 / s

    bend = (off + cp) // BM
    b_iota = lax.broadcasted_iota(jnp.int32, (1, 128), 1)
    acc = jnp.zeros((1, 128), jnp.int32)
    for e in range(NUM_EXPERTS):
        be = jnp.broadcast_to(bend[:, e:e + 1], (1, 128))
        acc = acc + (b_iota >= be).astype(jnp.int32)
    bexp = jnp.minimum(acc, NUM_EXPERTS - 1)
    bmeta_ref[...] = jnp.broadcast_to(bexp, (8, 128))


def _dispatch_body(x_hbm, pos_hbm, wrep_hbm, xg_hbm, wg_hbm,
                   xbuf, wbuf, p0, p1, sem):
    wid = lax.axis_index("s") * 2 + lax.axis_index("c")
    base = wid * TW
    pltpu.sync_copy(x_hbm.at[pl.ds(base, TW)], xbuf)
    pltpu.sync_copy(pos_hbm.at[0, pl.ds(base, TW)], p0)
    pltpu.sync_copy(pos_hbm.at[1, pl.ds(base, TW)], p1)
    cp0 = pltpu.async_copy(xbuf, xg_hbm.at[p0], sem)
    cp1 = pltpu.async_copy(xbuf, xg_hbm.at[p1], sem)
    pltpu.sync_copy(wrep_hbm.at[0, pl.ds(base, TW)], wbuf)
    cp2 = pltpu.async_copy(wbuf, wg_hbm.at[p0], sem)
    cp0.wait()
    cp1.wait()
    cp2.wait()
    pltpu.sync_copy(wrep_hbm.at[1, pl.ds(base, TW)], wbuf)
    pltpu.async_copy(wbuf, wg_hbm.at[p1], sem).wait()


def _gemm_body(be_ref, xg_ref, wg_ref, w1_ref, v1_ref, w2_ref, yg_ref):
    x = xg_ref[...]
    w1 = w1_ref[0]
    v1 = v1_ref[0]
    w2 = w2_ref[0]
    a = lax.dot_general(
        x, w1, (((1,), (1,)), ((), ())), preferred_element_type=jnp.float32)
    b = lax.dot_general(
        x, v1, (((1,), (1,)), ((), ())), preferred_element_type=jnp.float32)
    h = (a * lax.logistic(a)) * b
    y = lax.dot_general(
        h, w2, (((1,), (1,)), ((), ())), preferred_element_type=jnp.float32)
    yg_ref[...] = y * wg_ref[:, 0:1]


def _combine_body(yg_hbm, pos_hbm, out_hbm, p0, p1, y0, y1, sem):
    wid = lax.axis_index("s") * 2 + lax.axis_index("c")
    base = wid * TW
    for c in range(2):
        cbase = base + c * 32
        pltpu.sync_copy(pos_hbm.at[0, pl.ds(cbase, 32)], p0)
        pltpu.sync_copy(pos_hbm.at[1, pl.ds(cbase, 32)], p1)
        g0 = pltpu.async_copy(yg_hbm.at[p0], y0, sem)
        g1 = pltpu.async_copy(yg_hbm.at[p1], y1, sem)
        g0.wait()
        g1.wait()

        def add_row(j, _):
            for l in range(D_MODEL // 16):
                sl = pl.ds(l * 16, 16)
                y0[j, sl] = y0[j, sl] + y1[j, sl]
            return 0

        lax.fori_loop(0, 32, add_row, 0)
        pltpu.sync_copy(y0, out_hbm.at[pl.ds(cbase, 32)])


@jax.jit
def _run(x, w_router, ws, w2s):
    pos, wn, bmeta = pl.pallas_call(
        _route_body,
        out_shape=(
            jax.ShapeDtypeStruct((T, TOP_K), jnp.int32),
            jax.ShapeDtypeStruct((T, TOP_K), jnp.float32),
            jax.ShapeDtypeStruct((8, 128), jnp.int32),
        ),
    )(x, w_router)

    pos01 = pos.T
    wrep = jnp.broadcast_to(wn.T[:, :, None], (TOP_K, T, 128))
    be = bmeta[0, :NB]

    mesh = plsc.VectorSubcoreMesh(core_axis_name="c", subcore_axis_name="s")
    xg, wg = pl.kernel(
        _dispatch_body,
        mesh=mesh,
        out_type=(
            jax.ShapeDtypeStruct((NPAD, D_MODEL), jnp.float32),
            jax.ShapeDtypeStruct((NPAD, 128), jnp.float32),
        ),
        scratch_types=[
            pltpu.VMEM((TW, D_MODEL), jnp.float32),
            pltpu.VMEM((TW, 128), jnp.float32),
            pltpu.VMEM((TW,), jnp.int32),
            pltpu.VMEM((TW,), jnp.int32),
            pltpu.SemaphoreType.DMA,
        ],
    )(x, pos01, wrep)

    yg = pl.pallas_call(
        _gemm_body,
        grid_spec=pltpu.PrefetchScalarGridSpec(
            num_scalar_prefetch=1,
            grid=(NB,),
            in_specs=[
                pl.BlockSpec((BM, D_MODEL), lambda b, be: (b, 0)),
                pl.BlockSpec((BM, 128), lambda b, be: (b, 0)),
                pl.BlockSpec((1, D_FF, D_MODEL), lambda b, be: (be[b], 0, 0)),
                pl.BlockSpec((1, D_FF, D_MODEL), lambda b, be: (be[b], 1, 0)),
                pl.BlockSpec((1, D_MODEL, D_FF), lambda b, be: (be[b], 0, 0)),
            ],
            out_specs=pl.BlockSpec((BM, D_MODEL), lambda b, be: (b, 0)),
        ),
        out_shape=jax.ShapeDtypeStruct((NPAD, D_MODEL), jnp.float32),
        compiler_params=pltpu.CompilerParams(
            dimension_semantics=("arbitrary",)),
    )(be, xg, wg, ws, ws, w2s)

    out = pl.kernel(
        _combine_body,
        mesh=mesh,
        out_type=jax.ShapeDtypeStruct((T, D_MODEL), jnp.float32),
        scratch_types=[
            pltpu.VMEM((32,), jnp.int32),
            pltpu.VMEM((32,), jnp.int32),
            pltpu.VMEM((32, D_MODEL), jnp.float32),
            pltpu.VMEM((32, D_MODEL), jnp.float32),
            pltpu.SemaphoreType.DMA,
        ],
    )(yg, pos01)
    return out


def kernel(hidden_states, w_router, ws, w2s):
    b, s, d = hidden_states.shape
    x = hidden_states.reshape(s, d)
    out = _run(x, w_router, ws, w2s)
    return out.reshape(b, s, d)


# speedup vs baseline: 1.8564x; 1.4122x over previous
import functools

import jax
import jax.numpy as jnp
from jax import lax
from jax.experimental import pallas as pl
from jax.experimental.pallas import tpu as pltpu
from jax.experimental.pallas import tpu_sc as plsc

NUM_EXPERTS = 8
TOP_K = 2
D_MODEL = 1024
D_FF = 2048
T = 2048

BM = 256
NB = (T * TOP_K) // BM + NUM_EXPERTS
NPAD = NB * BM

NW = 32
TW = T // NW


def _route_body(x_ref, wr_ref, pos_ref, wn_ref, bmeta_ref):
    x = x_ref[...]
    wr = wr_ref[...]
    logits = lax.dot_general(
        x, wr, (((1,), (1,)), ((), ())), preferred_element_type=jnp.float32)
    m = jnp.max(logits, axis=1, keepdims=True)
    ex = jnp.exp(logits - m)
    p = ex / jnp.sum(ex, axis=1, keepdims=True)
    lane = lax.broadcasted_iota(jnp.int32, p.shape, 1)
    m0 = jnp.max(p, axis=1, keepdims=True)
    i0 = jnp.min(jnp.where(p == m0, lane, NUM_EXPERTS), axis=1, keepdims=True)
    p1m = jnp.where(lane == i0, -jnp.inf, p)
    m1 = jnp.max(p1m, axis=1, keepdims=True)
    i1 = jnp.min(jnp.where(p1m == m1, lane, NUM_EXPERTS), axis=1, keepdims=True)
    s = m0 + m1

    oh0 = (lane == i0).astype(jnp.int32)
    oh1 = (lane == i1).astype(jnp.int32)

    def cumsum_tok(a):
        sh = 1
        while sh < T:
            a = a + jnp.concatenate(
                [jnp.zeros((sh, NUM_EXPERTS), jnp.int32), a[:T - sh]], axis=0)
            sh *= 2
        return a

    c0 = cumsum_tok(oh0)
    c1 = cumsum_tok(oh1)
    tot0 = c0[T - 1:T, :]
    tot1 = c1[T - 1:T, :]
    cnt = tot0 + tot1
    cp = ((cnt + BM - 1) // BM) * BM

    ip = cp
    for sh in (1, 2, 4):
        ip = ip + jnp.concatenate(
            [jnp.zeros((1, sh), jnp.int32), ip[:, :NUM_EXPERTS - sh]], axis=1)
    off = ip - cp

    pos0 = jnp.sum(oh0 * (off + c0 - 1), axis=1, keepdims=True)
    pos1 = jnp.sum(oh1 * (off + tot0 + c1 - 1), axis=1, keepdims=True)
    pos_ref[:, 0:1] = pos0
    pos_ref[:, 1:2] = pos1
    wn_ref[:, 0:1] = m0 / s
    wn_ref[:, 1:2] = m1 / s

    bend = (off + cp) // BM
    b_iota = lax.broadcasted_iota(jnp.int32, (1, 128), 1)
    acc = jnp.zeros((1, 128), jnp.int32)
    for e in range(NUM_EXPERTS):
        be = jnp.broadcast_to(bend[:, e:e + 1], (1, 128))
        acc = acc + (b_iota >= be).astype(jnp.int32)
    bexp = jnp.minimum(acc, NUM_EXPERTS - 1)
    bmeta_ref[...] = jnp.broadcast_to(bexp, (8, 128))


def _dispatch_body(x_hbm, pos_hbm, wrep_hbm, xg_hbm, wg_hbm,
                   xbuf, wbuf, p0, p1, sem):
    wid = lax.axis_index("s") * 2 + lax.axis_index("c")
    base = wid * TW
    pltpu.sync_copy(x_hbm.at[pl.ds(base, TW)], xbuf)
    pltpu.sync_copy(pos_hbm.at[0, pl.ds(base, TW)], p0)
    pltpu.sync_copy(pos_hbm.at[1, pl.ds(base, TW)], p1)
    cp0 = pltpu.async_copy(xbuf, xg_hbm.at[p0], sem)
    cp1 = pltpu.async_copy(xbuf, xg_hbm.at[p1], sem)
    pltpu.sync_copy(wrep_hbm.at[0, pl.ds(base, TW)], wbuf)
    cp2 = pltpu.async_copy(wbuf, wg_hbm.at[p0], sem)
    cp0.wait()
    cp1.wait()
    cp2.wait()
    pltpu.sync_copy(wrep_hbm.at[1, pl.ds(base, TW)], wbuf)
    pltpu.async_copy(wbuf, wg_hbm.at[p1], sem).wait()


def _gemm_body(be_ref, xg_ref, wg_ref, w1_ref, v1_ref, w2_ref, yg_ref):
    x = xg_ref[...]
    w1 = w1_ref[0]
    v1 = v1_ref[0]
    w2 = w2_ref[0]
    a = lax.dot_general(
        x, w1, (((1,), (1,)), ((), ())), preferred_element_type=jnp.float32)
    b = lax.dot_general(
        x, v1, (((1,), (1,)), ((), ())), preferred_element_type=jnp.float32)
    h = (a * lax.logistic(a)) * b
    y = lax.dot_general(
        h, w2, (((1,), (1,)), ((), ())), preferred_element_type=jnp.float32)
    yg_ref[...] = y * wg_ref[:, 0:1]


def _combine_body(yg_hbm, pos_hbm, out_hbm, p0, p1, y0, y1, sem):
    wid = lax.axis_index("s") * 2 + lax.axis_index("c")
    base = wid * TW
    for c in range(2):
        cbase = base + c * 32
        pltpu.sync_copy(pos_hbm.at[0, pl.ds(cbase, 32)], p0)
        pltpu.sync_copy(pos_hbm.at[1, pl.ds(cbase, 32)], p1)
        g0 = pltpu.async_copy(yg_hbm.at[p0], y0, sem)
        g1 = pltpu.async_copy(yg_hbm.at[p1], y1, sem)
        g0.wait()
        g1.wait()

        def add_row(j, _):
            for l in range(D_MODEL // 16):
                sl = pl.ds(l * 16, 16)
                y0[j, sl] = y0[j, sl] + y1[j, sl]
            return 0

        lax.fori_loop(0, 32, add_row, 0)
        pltpu.sync_copy(y0, out_hbm.at[pl.ds(cbase, 32)])


@jax.jit
def _run(x, w_router, ws, w2s):
    pos, wn, bmeta = pl.pallas_call(
        _route_body,
        out_shape=(
            jax.ShapeDtypeStruct((T, TOP_K), jnp.int32),
            jax.ShapeDtypeStruct((T, TOP_K), jnp.float32),
            jax.ShapeDtypeStruct((8, 128), jnp.int32),
        ),
    )(x, w_router)

    pos01 = pos.T
    wrep = jnp.broadcast_to(wn.T[:, :, None], (TOP_K, T, 128))
    be = bmeta[0, :NB]

    mesh = plsc.VectorSubcoreMesh(core_axis_name="c", subcore_axis_name="s")
    xg, wg = pl.kernel(
        _dispatch_body,
        mesh=mesh,
        out_type=(
            jax.ShapeDtypeStruct((NPAD, D_MODEL), jnp.float32),
            jax.ShapeDtypeStruct((NPAD, 128), jnp.float32),
        ),
        scratch_types=[
            pltpu.VMEM((TW, D_MODEL), jnp.float32),
            pltpu.VMEM((TW, 128), jnp.float32),
            pltpu.VMEM((TW,), jnp.int32),
            pltpu.VMEM((TW,), jnp.int32),
            pltpu.SemaphoreType.DMA,
        ],
    )(x, pos01, wrep)

    yg = pl.pallas_call(
        _gemm_body,
        grid_spec=pltpu.PrefetchScalarGridSpec(
            num_scalar_prefetch=1,
            grid=(NB,),
            in_specs=[
                pl.BlockSpec((BM, D_MODEL), lambda b, be: (b, 0)),
                pl.BlockSpec((BM, 128), lambda b, be: (b, 0)),
                pl.BlockSpec((1, D_FF, D_MODEL), lambda b, be: (be[b], 0, 0)),
                pl.BlockSpec((1, D_FF, D_MODEL), lambda b, be: (be[b], 1, 0)),
                pl.BlockSpec((1, D_MODEL, D_FF), lambda b, be: (be[b], 0, 0)),
            ],
            out_specs=pl.BlockSpec((BM, D_MODEL), lambda b, be: (b, 0)),
        ),
        out_shape=jax.ShapeDtypeStruct((NPAD, D_MODEL), jnp.float32),
        compiler_params=pltpu.CompilerParams(
            dimension_semantics=("arbitrary",)),
    )(be, xg, wg, ws, ws, w2s)

    out = pl.kernel(
        _combine_body,
        mesh=mesh,
        out_type=jax.ShapeDtypeStruct((T, D_MODEL), jnp.float32),
        scratch_types=[
            pltpu.VMEM((32,), jnp.int32),
            pltpu.VMEM((32,), jnp.int32),
            pltpu.VMEM((32, D_MODEL), jnp.float32),
            pltpu.VMEM((32, D_MODEL), jnp.float32),
            pltpu.SemaphoreType.DMA,
        ],
    )(yg, pos01)
    return out


def kernel(hidden_states, w_router, ws, w2s):
    b, s, d = hidden_states.shape
    x = hidden_states.reshape(s, d)
    out = _run(x, w_router, ws, w2s)
    return out.reshape(b, s, d)


# speedup vs baseline: 1.8668x; 1.0056x over previous
import functools

import jax
import jax.numpy as jnp
from jax import lax
from jax.experimental import pallas as pl
from jax.experimental.pallas import tpu as pltpu
from jax.experimental.pallas import tpu_sc as plsc

NUM_EXPERTS = 8
TOP_K = 2
D_MODEL = 1024
D_FF = 2048
T = 2048

BM = 256
NB = (T * TOP_K) // BM + NUM_EXPERTS
NPAD = NB * BM

NW = 32
TW = T // NW


def _route_body(x_ref, wr_ref, pos_ref, wn_ref, bmeta_ref):
    x = x_ref[...]
    wr = wr_ref[...]
    logits = lax.dot_general(
        x, wr, (((1,), (1,)), ((), ())), preferred_element_type=jnp.float32)
    m = jnp.max(logits, axis=1, keepdims=True)
    ex = jnp.exp(logits - m)
    p = ex / jnp.sum(ex, axis=1, keepdims=True)
    lane = lax.broadcasted_iota(jnp.int32, p.shape, 1)
    m0 = jnp.max(p, axis=1, keepdims=True)
    i0 = jnp.min(jnp.where(p == m0, lane, NUM_EXPERTS), axis=1, keepdims=True)
    p1m = jnp.where(lane == i0, -jnp.inf, p)
    m1 = jnp.max(p1m, axis=1, keepdims=True)
    i1 = jnp.min(jnp.where(p1m == m1, lane, NUM_EXPERTS), axis=1, keepdims=True)
    s = m0 + m1

    oh0 = (lane == i0).astype(jnp.int32)
    oh1 = (lane == i1).astype(jnp.int32)

    def cumsum_tok(a):
        sh = 1
        while sh < T:
            a = a + jnp.concatenate(
                [jnp.zeros((sh, NUM_EXPERTS), jnp.int32), a[:T - sh]], axis=0)
            sh *= 2
        return a

    c0 = cumsum_tok(oh0)
    c1 = cumsum_tok(oh1)
    tot0 = c0[T - 1:T, :]
    tot1 = c1[T - 1:T, :]
    cnt = tot0 + tot1
    cp = ((cnt + BM - 1) // BM) * BM

    ip = cp
    for sh in (1, 2, 4):
        ip = ip + jnp.concatenate(
            [jnp.zeros((1, sh), jnp.int32), ip[:, :NUM_EXPERTS - sh]], axis=1)
    off = ip - cp

    pos0 = jnp.sum(oh0 * (off + c0 - 1), axis=1, keepdims=True)
    pos1 = jnp.sum(oh1 * (off + tot0 + c1 - 1), axis=1, keepdims=True)
    pos_ref[:, 0:1] = pos0
    pos_ref[:, 1:2] = pos1
    wn_ref[:, 0:1] = m0 / s
    wn_ref[:, 1:2] = m1 / s

    bend = (off + cp) // BM
    b_iota = lax.broadcasted_iota(jnp.int32, (1, 128), 1)
    acc = jnp.zeros((1, 128), jnp.int32)
    for e in range(NUM_EXPERTS):
        be = jnp.broadcast_to(bend[:, e:e + 1], (1, 128))
        acc = acc + (b_iota >= be).astype(jnp.int32)
    bexp = jnp.minimum(acc, NUM_EXPERTS - 1)
    bmeta_ref[...] = jnp.broadcast_to(bexp, (8, 128))


def _dispatch_body(x_hbm, pos_hbm, wrep_hbm, xg_hbm, wg_hbm,
                   xbuf, wbuf, p0, p1, sem):
    wid = lax.axis_index("s") * 2 + lax.axis_index("c")
    base = wid * TW
    pltpu.sync_copy(x_hbm.at[pl.ds(base, TW)], xbuf)
    pltpu.sync_copy(pos_hbm.at[0, pl.ds(base, TW)], p0)
    pltpu.sync_copy(pos_hbm.at[1, pl.ds(base, TW)], p1)
    cp0 = pltpu.async_copy(xbuf, xg_hbm.at[p0], sem)
    cp1 = pltpu.async_copy(xbuf, xg_hbm.at[p1], sem)
    pltpu.sync_copy(wrep_hbm.at[0, pl.ds(base, TW)], wbuf)
    cp2 = pltpu.async_copy(wbuf, wg_hbm.at[p0], sem)
    cp0.wait()
    cp1.wait()
    cp2.wait()
    pltpu.sync_copy(wrep_hbm.at[1, pl.ds(base, TW)], wbuf)
    pltpu.async_copy(wbuf, wg_hbm.at[p1], sem).wait()


def _gemm_body(be_ref, xg_ref, wg_ref, w_ref, w2_ref, yg_ref):
    x = xg_ref[...]
    w = w_ref[0]
    w2 = w2_ref[0]
    g = lax.dot_general(
        x, w, (((1,), (1,)), ((), ())), preferred_element_type=jnp.float32)
    a = g[:, :D_FF]
    b = g[:, D_FF:]
    h = (a * lax.logistic(a)) * b
    y = lax.dot_general(
        h, w2, (((1,), (1,)), ((), ())), preferred_element_type=jnp.float32)
    yg_ref[...] = y * wg_ref[:, 0:1]


def _combine_body(yg_hbm, pos_hbm, out_hbm, p0, p1, y0, y1, sem):
    wid = lax.axis_index("s") * 2 + lax.axis_index("c")
    base = wid * TW
    for c in range(2):
        cbase = base + c * 32
        pltpu.sync_copy(pos_hbm.at[0, pl.ds(cbase, 32)], p0)
        pltpu.sync_copy(pos_hbm.at[1, pl.ds(cbase, 32)], p1)
        g0 = pltpu.async_copy(yg_hbm.at[p0], y0, sem)
        g1 = pltpu.async_copy(yg_hbm.at[p1], y1, sem)
        g0.wait()
        g1.wait()

        def add_row(j, _):
            for l in range(D_MODEL // 16):
                sl = pl.ds(l * 16, 16)
                y0[j, sl] = y0[j, sl] + y1[j, sl]
            return 0

        lax.fori_loop(0, 32, add_row, 0)
        pltpu.sync_copy(y0, out_hbm.at[pl.ds(cbase, 32)])


@jax.jit
def _run(x, w_router, ws, w2s):
    pos, wn, bmeta = pl.pallas_call(
        _route_body,
        out_shape=(
            jax.ShapeDtypeStruct((T, TOP_K), jnp.int32),
            jax.ShapeDtypeStruct((T, TOP_K), jnp.float32),
            jax.ShapeDtypeStruct((8, 128), jnp.int32),
        ),
    )(x, w_router)

    pos01 = pos.T
    wrep = jnp.broadcast_to(wn.T[:, :, None], (TOP_K, T, 128))
    be = bmeta[0, :NB]

    mesh = plsc.VectorSubcoreMesh(core_axis_name="c", subcore_axis_name="s")
    xg, wg = pl.kernel(
        _dispatch_body,
        mesh=mesh,
        out_type=(
            jax.ShapeDtypeStruct((NPAD, D_MODEL), jnp.float32),
            jax.ShapeDtypeStruct((NPAD, 128), jnp.float32),
        ),
        scratch_types=[
            pltpu.VMEM((TW, D_MODEL), jnp.float32),
            pltpu.VMEM((TW, 128), jnp.float32),
            pltpu.VMEM((TW,), jnp.int32),
            pltpu.VMEM((TW,), jnp.int32),
            pltpu.SemaphoreType.DMA,
        ],
    )(x, pos01, wrep)

    yg = pl.pallas_call(
        _gemm_body,
        grid_spec=pltpu.PrefetchScalarGridSpec(
            num_scalar_prefetch=1,
            grid=(NB,),
            in_specs=[
                pl.BlockSpec((BM, D_MODEL), lambda b, be: (b, 0)),
                pl.BlockSpec((BM, 128), lambda b, be: (b, 0)),
                pl.BlockSpec((1, 2 * D_FF, D_MODEL), lambda b, be: (be[b], 0, 0)),
                pl.BlockSpec((1, D_MODEL, D_FF), lambda b, be: (be[b], 0, 0)),
            ],
            out_specs=pl.BlockSpec((BM, D_MODEL), lambda b, be: (b, 0)),
        ),
        out_shape=jax.ShapeDtypeStruct((NPAD, D_MODEL), jnp.float32),
        compiler_params=pltpu.CompilerParams(
            dimension_semantics=("arbitrary",)),
    )(be, xg, wg, ws, w2s)

    out = pl.kernel(
        _combine_body,
        mesh=mesh,
        out_type=jax.ShapeDtypeStruct((T, D_MODEL), jnp.float32),
        scratch_types=[
            pltpu.VMEM((32,), jnp.int32),
            pltpu.VMEM((32,), jnp.int32),
            pltpu.VMEM((32, D_MODEL), jnp.float32),
            pltpu.VMEM((32, D_MODEL), jnp.float32),
            pltpu.SemaphoreType.DMA,
        ],
    )(yg, pos01)
    return out


def kernel(hidden_states, w_router, ws, w2s):
    b, s, d = hidden_states.shape
    x = hidden_states.reshape(s, d)
    out = _run(x, w_router, ws, w2s)
    return out.reshape(b, s, d)


# speedup vs baseline: 2.3221x; 1.2439x over previous
import jax
import jax.numpy as jnp
from jax import lax
from jax.experimental import pallas as pl
from jax.experimental.pallas import tpu as pltpu
from jax.experimental.pallas import tpu_sc as plsc

NUM_EXPERTS = 8
TOP_K = 2
D_MODEL = 1024
D_FF = 2048
T = 2048

BM = 256
NB = (T * TOP_K) // BM + NUM_EXPERTS
NPAD = NB * BM

NW = 32
TW = T // NW


def _route_body(x_ref, wr_ref, pos_ref, w0_ref, w1_ref, bmeta_ref):
    x = x_ref[...]
    wr = wr_ref[...]
    logits = lax.dot_general(
        x, wr, (((1,), (1,)), ((), ())), preferred_element_type=jnp.float32)
    m = jnp.max(logits, axis=1, keepdims=True)
    ex = jnp.exp(logits - m)
    p = ex / jnp.sum(ex, axis=1, keepdims=True)
    lane = lax.broadcasted_iota(jnp.int32, p.shape, 1)
    m0 = jnp.max(p, axis=1, keepdims=True)
    i0 = jnp.min(jnp.where(p == m0, lane, NUM_EXPERTS), axis=1, keepdims=True)
    p1m = jnp.where(lane == i0, -jnp.inf, p)
    m1 = jnp.max(p1m, axis=1, keepdims=True)
    i1 = jnp.min(jnp.where(p1m == m1, lane, NUM_EXPERTS), axis=1, keepdims=True)
    s = m0 + m1

    oh0 = (lane == i0).astype(jnp.int32)
    oh1 = (lane == i1).astype(jnp.int32)

    def cumsum_tok(a):
        sh = 1
        while sh < T:
            a = a + jnp.concatenate(
                [jnp.zeros((sh, NUM_EXPERTS), jnp.int32), a[:T - sh]], axis=0)
            sh *= 2
        return a

    c0 = cumsum_tok(oh0)
    c1 = cumsum_tok(oh1)
    tot0 = c0[T - 1:T, :]
    tot1 = c1[T - 1:T, :]
    cnt = tot0 + tot1
    cp = ((cnt + BM - 1) // BM) * BM

    ip = cp
    for sh in (1, 2, 4):
        ip = ip + jnp.concatenate(
            [jnp.zeros((1, sh), jnp.int32), ip[:, :NUM_EXPERTS - sh]], axis=1)
    off = ip - cp

    pos0 = jnp.sum(oh0 * (off + c0 - 1), axis=1, keepdims=True)
    pos1 = jnp.sum(oh1 * (off + tot0 + c1 - 1), axis=1, keepdims=True)
    pos_ref[:, 0:1] = pos0
    pos_ref[:, 1:2] = pos1
    w0_ref[...] = jnp.broadcast_to(m0 / s, (T, 128))
    w1_ref[...] = jnp.broadcast_to(m1 / s, (T, 128))

    bend = (off + cp) // BM
    bstart = off // BM
    present = (cp > 0).astype(jnp.int32)
    rank = jnp.zeros((1, NUM_EXPERTS), jnp.int32)
    racc = present
    for sh in (1, 2, 4):
        racc = racc + jnp.concatenate(
            [jnp.zeros((1, sh), jnp.int32), racc[:, :NUM_EXPERTS - sh]], axis=1)
    rank = racc - present
    num_runs = jnp.sum(present, axis=1, keepdims=True)
    nused = ip[:, NUM_EXPERTS - 1:NUM_EXPERTS] // BM

    lane128 = lax.broadcasted_iota(jnp.int32, (1, 128), 1)
    bexp = jnp.zeros((1, 128), jnp.int32)
    rs = jnp.zeros((1, 128), jnp.int32)
    for e in range(NUM_EXPERTS):
        be_e = jnp.broadcast_to(bend[:, e:e + 1], (1, 128))
        bexp = bexp + (lane128 >= be_e).astype(jnp.int32)
        pr_e = jnp.broadcast_to(present[:, e:e + 1], (1, 128))
        rk_e = jnp.broadcast_to(rank[:, e:e + 1], (1, 128))
        bs_e = jnp.broadcast_to(bstart[:, e:e + 1], (1, 128))
        rs = rs + jnp.where((lane128 == rk_e) & (pr_e > 0), bs_e, 0)
    bexp = jnp.minimum(bexp, NUM_EXPERTS - 1)
    nr_b = jnp.broadcast_to(num_runs, (1, 128))
    rs = jnp.where(lane128 >= nr_b, NB, rs)
    runid = jnp.zeros((1, 128), jnp.int32)
    for r in range(1, NUM_EXPERTS + 1):
        rs_r = jnp.broadcast_to(rs[:, r:r + 1], (1, 128))
        runid = runid + (lane128 >= rs_r).astype(jnp.int32)
    nu_b = jnp.broadcast_to(nused, (1, 128))
    bmeta_ref[...] = jnp.concatenate(
        [bexp, runid, rs, nu_b, jnp.zeros((4, 128), jnp.int32)], axis=0)


def _dispatch_body(x_hbm, pos_hbm, xg_hbm, xbuf, p0, p1, sem):
    wid = lax.axis_index("s") * 2 + lax.axis_index("c")
    base = wid * TW
    pltpu.sync_copy(x_hbm.at[pl.ds(base, TW)], xbuf)
    pltpu.sync_copy(pos_hbm.at[0, pl.ds(base, TW)], p0)
    pltpu.sync_copy(pos_hbm.at[1, pl.ds(base, TW)], p1)
    cp0 = pltpu.async_copy(xbuf, xg_hbm.at[p0], sem)
    cp1 = pltpu.async_copy(xbuf, xg_hbm.at[p1], sem)
    cp0.wait()
    cp1.wait()


def _gemm_body(meta_ref, xg_ref, ws_hbm, w2s_hbm, yg_ref,
               wbuf, w2buf, wsem, w2sem):
    b = pl.program_id(0)
    nused = meta_ref[3, 0]
    rid = meta_ref[1, b]
    rs_cur = meta_ref[2, rid]
    rs_nxt = meta_ref[2, rid + 1]
    slot = lax.rem(rid, 2)
    nslot = lax.rem(rid + 1, 2)

    @pl.when(b == 0)
    def _():
        e0 = meta_ref[0, 0]
        pltpu.make_async_copy(ws_hbm.at[e0], wbuf.at[0], wsem.at[0]).start()
        pltpu.make_async_copy(w2s_hbm.at[e0], w2buf.at[0], w2sem.at[0]).start()

    @pl.when((b == rs_cur) & (b < nused))
    def _():
        @pl.when(rs_nxt < nused)
        def _():
            en = meta_ref[0, rs_nxt]
            pltpu.make_async_copy(
                ws_hbm.at[en], wbuf.at[nslot], wsem.at[nslot]).start()
            pltpu.make_async_copy(
                w2s_hbm.at[en], w2buf.at[nslot], w2sem.at[nslot]).start()
        ec = meta_ref[0, b]
        pltpu.make_async_copy(ws_hbm.at[ec], wbuf.at[slot], wsem.at[slot]).wait()
        pltpu.make_async_copy(
            w2s_hbm.at[ec], w2buf.at[slot], w2sem.at[slot]).wait()

    @pl.when(b < nused)
    def _():
        x = xg_ref[...]
        y = jnp.zeros((BM, D_MODEL), jnp.float32)
        for f in range(2):
            w1f = wbuf[slot, pl.ds(f * D_MODEL, D_MODEL), :]
            v1f = wbuf[slot, pl.ds(D_FF + f * D_MODEL, D_MODEL), :]
            w2f = w2buf[slot, :, pl.ds(f * D_MODEL, D_MODEL)]
            a = lax.dot_general(x, w1f, (((1,), (1,)), ((), ())),
                                preferred_element_type=jnp.float32)
            bb = lax.dot_general(x, v1f, (((1,), (1,)), ((), ())),
                                 preferred_element_type=jnp.float32)
            h = (a * lax.logistic(a)) * bb
            y = y + lax.dot_general(h, w2f, (((1,), (1,)), ((), ())),
                                    preferred_element_type=jnp.float32)
        yg_ref[...] = y


def _combine_body(yg_hbm, pos_hbm, wr0_hbm, wr1_hbm, out_hbm,
                  p0, p1, y0, y1, wb0, wb1, sem):
    wid = lax.axis_index("s") * 2 + lax.axis_index("c")
    base = wid * TW
    for c in range(2):
        cbase = base + c * 32
        pltpu.sync_copy(pos_hbm.at[0, pl.ds(cbase, 32)], p0)
        pltpu.sync_copy(pos_hbm.at[1, pl.ds(cbase, 32)], p1)
        g0 = pltpu.async_copy(yg_hbm.at[p0], y0, sem)
        g1 = pltpu.async_copy(yg_hbm.at[p1], y1, sem)
        pltpu.sync_copy(wr0_hbm.at[pl.ds(cbase, 32)], wb0)
        pltpu.sync_copy(wr1_hbm.at[pl.ds(cbase, 32)], wb1)
        g0.wait()
        g1.wait()

        def add_row(j, _):
            wv0 = wb0[j, pl.ds(0, 16)]
            wv1 = wb1[j, pl.ds(0, 16)]
            for l in range(D_MODEL // 16):
                sl = pl.ds(l * 16, 16)
                y0[j, sl] = y0[j, sl] * wv0 + y1[j, sl] * wv1
            return 0

        lax.fori_loop(0, 32, add_row, 0)
        pltpu.sync_copy(y0, out_hbm.at[pl.ds(cbase, 32)])


@jax.jit
def _run(x, w_router, ws, w2s):
    pos, wr0, wr1, bmeta = pl.pallas_call(
        _route_body,
        out_shape=(
            jax.ShapeDtypeStruct((T, TOP_K), jnp.int32),
            jax.ShapeDtypeStruct((T, 128), jnp.float32),
            jax.ShapeDtypeStruct((T, 128), jnp.float32),
            jax.ShapeDtypeStruct((8, 128), jnp.int32),
        ),
    )(x, w_router)

    pos01 = pos.T

    mesh = plsc.VectorSubcoreMesh(core_axis_name="c", subcore_axis_name="s")
    xg = pl.kernel(
        _dispatch_body,
        mesh=mesh,
        out_type=jax.ShapeDtypeStruct((NPAD, D_MODEL), jnp.float32),
        scratch_types=[
            pltpu.VMEM((TW, D_MODEL), jnp.float32),
            pltpu.VMEM((TW,), jnp.int32),
            pltpu.VMEM((TW,), jnp.int32),
            pltpu.SemaphoreType.DMA,
        ],
    )(x, pos01)

    yg = pl.pallas_call(
        _gemm_body,
        grid_spec=pltpu.PrefetchScalarGridSpec(
            num_scalar_prefetch=1,
            grid=(NB,),
            in_specs=[
                pl.BlockSpec((BM, D_MODEL), lambda b, m: (b, 0)),
                pl.BlockSpec(memory_space=pl.ANY),
                pl.BlockSpec(memory_space=pl.ANY),
            ],
            out_specs=pl.BlockSpec((BM, D_MODEL), lambda b, m: (b, 0)),
            scratch_shapes=[
                pltpu.VMEM((2, 2 * D_FF, D_MODEL), jnp.float32),
                pltpu.VMEM((2, D_MODEL, D_FF), jnp.float32),
                pltpu.SemaphoreType.DMA((2,)),
                pltpu.SemaphoreType.DMA((2,)),
            ],
        ),
        out_shape=jax.ShapeDtypeStruct((NPAD, D_MODEL), jnp.float32),
        compiler_params=pltpu.CompilerParams(
            dimension_semantics=("arbitrary",)),
    )(bmeta, xg, ws, w2s)

    out = pl.kernel(
        _combine_body,
        mesh=mesh,
        out_type=jax.ShapeDtypeStruct((T, D_MODEL), jnp.float32),
        scratch_types=[
            pltpu.VMEM((32,), jnp.int32),
            pltpu.VMEM((32,), jnp.int32),
            pltpu.VMEM((32, D_MODEL), jnp.float32),
            pltpu.VMEM((32, D_MODEL), jnp.float32),
            pltpu.VMEM((32, 128), jnp.float32),
            pltpu.VMEM((32, 128), jnp.float32),
            pltpu.SemaphoreType.DMA,
        ],
    )(yg, pos01, wr0, wr1)
    return out


def kernel(hidden_states, w_router, ws, w2s):
    b, s, d = hidden_states.shape
    x = hidden_states.reshape(s, d)
    out = _run(x, w_router, ws, w2s)
    return out.reshape(b, s, d)


# speedup vs baseline: 2.3523x; 1.0130x over previous
import jax
import jax.numpy as jnp
from jax import lax
from jax.experimental import pallas as pl
from jax.experimental.pallas import tpu as pltpu
from jax.experimental.pallas import tpu_sc as plsc

NUM_EXPERTS = 8
TOP_K = 2
D_MODEL = 1024
D_FF = 2048
T = 2048

BM = 256
NB = (T * TOP_K) // BM + NUM_EXPERTS
NPAD = NB * BM

NW = 32
TW = T // NW


def _route_body(x_ref, wr_ref, pos_ref, w0_ref, w1_ref, bmeta_ref):
    x = x_ref[...]
    wr = wr_ref[...]
    logits = lax.dot_general(
        x, wr, (((1,), (1,)), ((), ())), preferred_element_type=jnp.float32)
    m = jnp.max(logits, axis=1, keepdims=True)
    ex = jnp.exp(logits - m)
    p = ex / jnp.sum(ex, axis=1, keepdims=True)
    lane = lax.broadcasted_iota(jnp.int32, p.shape, 1)
    m0 = jnp.max(p, axis=1, keepdims=True)
    i0 = jnp.min(jnp.where(p == m0, lane, NUM_EXPERTS), axis=1, keepdims=True)
    p1m = jnp.where(lane == i0, -jnp.inf, p)
    m1 = jnp.max(p1m, axis=1, keepdims=True)
    i1 = jnp.min(jnp.where(p1m == m1, lane, NUM_EXPERTS), axis=1, keepdims=True)
    s = m0 + m1

    oh0 = (lane == i0).astype(jnp.int32)
    oh1 = (lane == i1).astype(jnp.int32)

    def cumsum_tok(a):
        sh = 1
        while sh < T:
            a = a + jnp.concatenate(
                [jnp.zeros((sh, NUM_EXPERTS), jnp.int32), a[:T - sh]], axis=0)
            sh *= 2
        return a

    c0 = cumsum_tok(oh0)
    c1 = cumsum_tok(oh1)
    tot0 = c0[T - 1:T, :]
    tot1 = c1[T - 1:T, :]
    cnt = tot0 + tot1
    cp = ((cnt + BM - 1) // BM) * BM

    ip = cp
    for sh in (1, 2, 4):
        ip = ip + jnp.concatenate(
            [jnp.zeros((1, sh), jnp.int32), ip[:, :NUM_EXPERTS - sh]], axis=1)
    off = ip - cp

    pos0 = jnp.sum(oh0 * (off + c0 - 1), axis=1, keepdims=True)
    pos1 = jnp.sum(oh1 * (off + tot0 + c1 - 1), axis=1, keepdims=True)
    pos_ref[:, 0:1] = pos0
    pos_ref[:, 1:2] = pos1
    w0_ref[...] = jnp.broadcast_to(m0 / s, (T, 128))
    w1_ref[...] = jnp.broadcast_to(m1 / s, (T, 128))

    bend = (off + cp) // BM
    bstart = off // BM
    present = (cp > 0).astype(jnp.int32)
    rank = jnp.zeros((1, NUM_EXPERTS), jnp.int32)
    racc = present
    for sh in (1, 2, 4):
        racc = racc + jnp.concatenate(
            [jnp.zeros((1, sh), jnp.int32), racc[:, :NUM_EXPERTS - sh]], axis=1)
    rank = racc - present
    num_runs = jnp.sum(present, axis=1, keepdims=True)
    nused = ip[:, NUM_EXPERTS - 1:NUM_EXPERTS] // BM

    lane128 = lax.broadcasted_iota(jnp.int32, (1, 128), 1)
    bexp = jnp.zeros((1, 128), jnp.int32)
    rs = jnp.zeros((1, 128), jnp.int32)
    for e in range(NUM_EXPERTS):
        be_e = jnp.broadcast_to(bend[:, e:e + 1], (1, 128))
        bexp = bexp + (lane128 >= be_e).astype(jnp.int32)
        pr_e = jnp.broadcast_to(present[:, e:e + 1], (1, 128))
        rk_e = jnp.broadcast_to(rank[:, e:e + 1], (1, 128))
        bs_e = jnp.broadcast_to(bstart[:, e:e + 1], (1, 128))
        rs = rs + jnp.where((lane128 == rk_e) & (pr_e > 0), bs_e, 0)
    bexp = jnp.minimum(bexp, NUM_EXPERTS - 1)
    nr_b = jnp.broadcast_to(num_runs, (1, 128))
    rs = jnp.where(lane128 >= nr_b, NB, rs)
    runid = jnp.zeros((1, 128), jnp.int32)
    for r in range(1, NUM_EXPERTS + 1):
        rs_r = jnp.broadcast_to(rs[:, r:r + 1], (1, 128))
        runid = runid + (lane128 >= rs_r).astype(jnp.int32)
    nu_b = jnp.broadcast_to(nused, (1, 128))
    bmeta_ref[...] = jnp.concatenate(
        [bexp, runid, rs, nu_b, jnp.zeros((4, 128), jnp.int32)], axis=0)


def _dispatch_body(x_hbm, pos_hbm, xg_hbm, xbuf, p0, p1, sem):
    wid = lax.axis_index("s") * 2 + lax.axis_index("c")
    base = wid * TW
    pltpu.sync_copy(x_hbm.at[pl.ds(base, TW)], xbuf)
    pltpu.sync_copy(pos_hbm.at[0, pl.ds(base, TW)], p0)
    pltpu.sync_copy(pos_hbm.at[1, pl.ds(base, TW)], p1)
    cp0 = pltpu.async_copy(xbuf, xg_hbm.at[p0], sem)
    cp1 = pltpu.async_copy(xbuf, xg_hbm.at[p1], sem)
    cp0.wait()
    cp1.wait()


def _gemm_body(meta_ref, xg_ref, ws_hbm, w2s_hbm, yg_ref,
               wbuf, w2buf, wsem, w2sem):
    b = pl.program_id(0)
    nused = meta_ref[3, 0]
    rid = meta_ref[1, b]
    rs_cur = meta_ref[2, rid]
    rs_nxt = meta_ref[2, rid + 1]
    slot = lax.rem(rid, 2)
    nslot = lax.rem(rid + 1, 2)

    def _w_copies(e, sl):
        cs = []
        for hh in range(4):
            cs.append(pltpu.make_async_copy(
                ws_hbm.at[e, pl.ds(hh * D_FF // 2, D_FF // 2)],
                wbuf.at[sl, pl.ds(hh * D_FF // 2, D_FF // 2)],
                wsem.at[sl]))
        for hh in range(2):
            cs.append(pltpu.make_async_copy(
                w2s_hbm.at[e, pl.ds(hh * D_MODEL // 2, D_MODEL // 2)],
                w2buf.at[sl, pl.ds(hh * D_MODEL // 2, D_MODEL // 2)],
                w2sem.at[sl]))
        return cs

    @pl.when(b == 0)
    def _():
        for c in _w_copies(meta_ref[0, 0], 0):
            c.start()

    @pl.when((b == rs_cur) & (b < nused))
    def _():
        @pl.when(rs_nxt < nused)
        def _():
            for c in _w_copies(meta_ref[0, rs_nxt], nslot):
                c.start()
        for c in _w_copies(meta_ref[0, b], slot):
            c.wait()

    @pl.when(b < nused)
    def _():
        x = xg_ref[...]
        y = jnp.zeros((BM, D_MODEL), jnp.float32)
        for f in range(2):
            w1f = wbuf[slot, pl.ds(f * D_MODEL, D_MODEL), :]
            v1f = wbuf[slot, pl.ds(D_FF + f * D_MODEL, D_MODEL), :]
            w2f = w2buf[slot, :, pl.ds(f * D_MODEL, D_MODEL)]
            a = lax.dot_general(x, w1f, (((1,), (1,)), ((), ())),
                                preferred_element_type=jnp.float32)
            bb = lax.dot_general(x, v1f, (((1,), (1,)), ((), ())),
                                 preferred_element_type=jnp.float32)
            h = (a * lax.logistic(a)) * bb
            y = y + lax.dot_general(h, w2f, (((1,), (1,)), ((), ())),
                                    preferred_element_type=jnp.float32)
        yg_ref[...] = y


def _combine_body(yg_hbm, pos_hbm, wr0_hbm, wr1_hbm, out_hbm,
                  p0, p1, y0, y1, wb0, wb1, sem):
    wid = lax.axis_index("s") * 2 + lax.axis_index("c")
    base = wid * TW
    for c in range(2):
        cbase = base + c * 32
        pltpu.sync_copy(pos_hbm.at[0, pl.ds(cbase, 32)], p0)
        pltpu.sync_copy(pos_hbm.at[1, pl.ds(cbase, 32)], p1)
        g0 = pltpu.async_copy(yg_hbm.at[p0], y0, sem)
        g1 = pltpu.async_copy(yg_hbm.at[p1], y1, sem)
        pltpu.sync_copy(wr0_hbm.at[pl.ds(cbase, 32)], wb0)
        pltpu.sync_copy(wr1_hbm.at[pl.ds(cbase, 32)], wb1)
        g0.wait()
        g1.wait()

        def add_row(j, _):
            wv0 = wb0[j, pl.ds(0, 16)]
            wv1 = wb1[j, pl.ds(0, 16)]
            for l in range(D_MODEL // 16):
                sl = pl.ds(l * 16, 16)
                y0[j, sl] = y0[j, sl] * wv0 + y1[j, sl] * wv1
            return 0

        lax.fori_loop(0, 32, add_row, 0)
        pltpu.sync_copy(y0, out_hbm.at[pl.ds(cbase, 32)])


@jax.jit
def _run(x, w_router, ws, w2s):
    pos, wr0, wr1, bmeta = pl.pallas_call(
        _route_body,
        out_shape=(
            jax.ShapeDtypeStruct((T, TOP_K), jnp.int32),
            jax.ShapeDtypeStruct((T, 128), jnp.float32),
            jax.ShapeDtypeStruct((T, 128), jnp.float32),
            jax.ShapeDtypeStruct((8, 128), jnp.int32),
        ),
    )(x, w_router)

    pos01 = pos.T

    mesh = plsc.VectorSubcoreMesh(core_axis_name="c", subcore_axis_name="s")
    xg = pl.kernel(
        _dispatch_body,
        mesh=mesh,
        out_type=jax.ShapeDtypeStruct((NPAD, D_MODEL), jnp.float32),
        scratch_types=[
            pltpu.VMEM((TW, D_MODEL), jnp.float32),
            pltpu.VMEM((TW,), jnp.int32),
            pltpu.VMEM((TW,), jnp.int32),
            pltpu.SemaphoreType.DMA,
        ],
    )(x, pos01)

    yg = pl.pallas_call(
        _gemm_body,
        grid_spec=pltpu.PrefetchScalarGridSpec(
            num_scalar_prefetch=1,
            grid=(NB,),
            in_specs=[
                pl.BlockSpec((BM, D_MODEL), lambda b, m: (b, 0)),
                pl.BlockSpec(memory_space=pl.ANY),
                pl.BlockSpec(memory_space=pl.ANY),
            ],
            out_specs=pl.BlockSpec((BM, D_MODEL), lambda b, m: (b, 0)),
            scratch_shapes=[
                pltpu.VMEM((2, 2 * D_FF, D_MODEL), jnp.float32),
                pltpu.VMEM((2, D_MODEL, D_FF), jnp.float32),
                pltpu.SemaphoreType.DMA((2,)),
                pltpu.SemaphoreType.DMA((2,)),
            ],
        ),
        out_shape=jax.ShapeDtypeStruct((NPAD, D_MODEL), jnp.float32),
        compiler_params=pltpu.CompilerParams(
            dimension_semantics=("arbitrary",)),
    )(bmeta, xg, ws, w2s)

    out = pl.kernel(
        _combine_body,
        mesh=mesh,
        out_type=jax.ShapeDtypeStruct((T, D_MODEL), jnp.float32),
        scratch_types=[
            pltpu.VMEM((32,), jnp.int32),
            pltpu.VMEM((32,), jnp.int32),
            pltpu.VMEM((32, D_MODEL), jnp.float32),
            pltpu.VMEM((32, D_MODEL), jnp.float32),
            pltpu.VMEM((32, 128), jnp.float32),
            pltpu.VMEM((32, 128), jnp.float32),
            pltpu.SemaphoreType.DMA,
        ],
    )(yg, pos01, wr0, wr1)
    return out


def kernel(hidden_states, w_router, ws, w2s):
    b, s, d = hidden_states.shape
    x = hidden_states.reshape(s, d)
    out = _run(x, w_router, ws, w2s)
    return out.reshape(b, s, d)


# speedup vs baseline: 2.3848x; 1.0138x over previous
import jax
import jax.numpy as jnp
from jax import lax
from jax.experimental import pallas as pl
from jax.experimental.pallas import tpu as pltpu
from jax.experimental.pallas import tpu_sc as plsc

NUM_EXPERTS = 8
TOP_K = 2
D_MODEL = 1024
D_FF = 2048
T = 2048

BM = 256
NB = (T * TOP_K) // BM + NUM_EXPERTS
NPAD = NB * BM

NW = 32
TW = T // NW


def _route_body(x_ref, wr_ref, pos_ref, w0_ref, w1_ref, bmeta_ref):
    x = x_ref[...]
    wr = wr_ref[...]
    logits = lax.dot_general(
        x, wr, (((1,), (1,)), ((), ())), preferred_element_type=jnp.float32)
    m = jnp.max(logits, axis=1, keepdims=True)
    ex = jnp.exp(logits - m)
    p = ex / jnp.sum(ex, axis=1, keepdims=True)
    lane = lax.broadcasted_iota(jnp.int32, p.shape, 1)
    m0 = jnp.max(p, axis=1, keepdims=True)
    i0 = jnp.min(jnp.where(p == m0, lane, NUM_EXPERTS), axis=1, keepdims=True)
    p1m = jnp.where(lane == i0, -jnp.inf, p)
    m1 = jnp.max(p1m, axis=1, keepdims=True)
    i1 = jnp.min(jnp.where(p1m == m1, lane, NUM_EXPERTS), axis=1, keepdims=True)
    s = m0 + m1

    oh0 = (lane == i0).astype(jnp.int32)
    oh1 = (lane == i1).astype(jnp.int32)

    def cumsum_tok(a):
        sh = 1
        while sh < T:
            a = a + jnp.concatenate(
                [jnp.zeros((sh, NUM_EXPERTS), jnp.int32), a[:T - sh]], axis=0)
            sh *= 2
        return a

    c0 = cumsum_tok(oh0)
    c1 = cumsum_tok(oh1)
    tot0 = c0[T - 1:T, :]
    tot1 = c1[T - 1:T, :]
    cnt = tot0 + tot1
    cp = ((cnt + BM - 1) // BM) * BM

    ip = cp
    for sh in (1, 2, 4):
        ip = ip + jnp.concatenate(
            [jnp.zeros((1, sh), jnp.int32), ip[:, :NUM_EXPERTS - sh]], axis=1)
    off = ip - cp

    pos0 = jnp.sum(oh0 * (off + c0 - 1), axis=1, keepdims=True)
    pos1 = jnp.sum(oh1 * (off + tot0 + c1 - 1), axis=1, keepdims=True)
    pos_ref[:, 0:1] = pos0
    pos_ref[:, 1:2] = pos1
    w0_ref[...] = jnp.broadcast_to(m0 / s, (T, 128))
    w1_ref[...] = jnp.broadcast_to(m1 / s, (T, 128))

    bend = (off + cp) // BM
    bstart = off // BM
    present = (cp > 0).astype(jnp.int32)
    rank = jnp.zeros((1, NUM_EXPERTS), jnp.int32)
    racc = present
    for sh in (1, 2, 4):
        racc = racc + jnp.concatenate(
            [jnp.zeros((1, sh), jnp.int32), racc[:, :NUM_EXPERTS - sh]], axis=1)
    rank = racc - present
    num_runs = jnp.sum(present, axis=1, keepdims=True)
    nused = ip[:, NUM_EXPERTS - 1:NUM_EXPERTS] // BM

    lane128 = lax.broadcasted_iota(jnp.int32, (1, 128), 1)
    bexp = jnp.zeros((1, 128), jnp.int32)
    rs = jnp.zeros((1, 128), jnp.int32)
    for e in range(NUM_EXPERTS):
        be_e = jnp.broadcast_to(bend[:, e:e + 1], (1, 128))
        bexp = bexp + (lane128 >= be_e).astype(jnp.int32)
        pr_e = jnp.broadcast_to(present[:, e:e + 1], (1, 128))
        rk_e = jnp.broadcast_to(rank[:, e:e + 1], (1, 128))
        bs_e = jnp.broadcast_to(bstart[:, e:e + 1], (1, 128))
        rs = rs + jnp.where((lane128 == rk_e) & (pr_e > 0), bs_e, 0)
    bexp = jnp.minimum(bexp, NUM_EXPERTS - 1)
    nr_b = jnp.broadcast_to(num_runs, (1, 128))
    rs = jnp.where(lane128 >= nr_b, NB, rs)
    runid = jnp.zeros((1, 128), jnp.int32)
    for r in range(1, NUM_EXPERTS + 1):
        rs_r = jnp.broadcast_to(rs[:, r:r + 1], (1, 128))
        runid = runid + (lane128 >= rs_r).astype(jnp.int32)
    nu_b = jnp.broadcast_to(nused, (1, 128))
    bmeta_ref[...] = jnp.concatenate(
        [bexp, runid, rs, nu_b, jnp.zeros((4, 128), jnp.int32)], axis=0)


def _dispatch_body(x_hbm, pos_hbm, xg_hbm, xbuf, p0, p1, sem):
    wid = lax.axis_index("s") * 2 + lax.axis_index("c")
    base = wid * TW
    pltpu.sync_copy(x_hbm.at[pl.ds(base, TW)], xbuf)
    pltpu.sync_copy(pos_hbm.at[0, pl.ds(base, TW)], p0)
    pltpu.sync_copy(pos_hbm.at[1, pl.ds(base, TW)], p1)
    cp0 = pltpu.async_copy(xbuf, xg_hbm.at[p0], sem)
    cp1 = pltpu.async_copy(xbuf, xg_hbm.at[p1], sem)
    cp0.wait()
    cp1.wait()


def _gemm_body(meta_ref, xg_ref, ws_hbm, w2s_hbm, yg_ref,
               wbuf, w2buf, wsem, w2sem):
    b = pl.program_id(0)
    nused = meta_ref[3, 0]
    rid = meta_ref[1, b]
    rs_cur = meta_ref[2, rid]
    rs_nxt = meta_ref[2, rid + 1]
    slot = lax.rem(rid, 2)
    nslot = lax.rem(rid + 1, 2)

    def _w_copies(e, sl):
        cs = []
        for hh in range(4):
            cs.append(pltpu.make_async_copy(
                ws_hbm.at[e, pl.ds(hh * D_FF // 2, D_FF // 2)],
                wbuf.at[sl, pl.ds(hh * D_FF // 2, D_FF // 2)],
                wsem.at[sl]))
        for hh in range(2):
            cs.append(pltpu.make_async_copy(
                w2s_hbm.at[e, pl.ds(hh * D_MODEL // 2, D_MODEL // 2)],
                w2buf.at[sl, pl.ds(hh * D_MODEL // 2, D_MODEL // 2)],
                w2sem.at[sl]))
        return cs

    @pl.when(b == 0)
    def _():
        for c in _w_copies(meta_ref[0, 0], 0):
            c.start()

    @pl.when((b == rs_cur) & (b < nused))
    def _():
        @pl.when(rs_nxt < nused)
        def _():
            for c in _w_copies(meta_ref[0, rs_nxt], nslot):
                c.start()
        for c in _w_copies(meta_ref[0, b], slot):
            c.wait()

    @pl.when(b < nused)
    def _():
        x = xg_ref[...]
        y = jnp.zeros((BM, D_MODEL), jnp.float32)
        for f in range(2):
            w1f = wbuf[slot, pl.ds(f * D_MODEL, D_MODEL), :]
            v1f = wbuf[slot, pl.ds(D_FF + f * D_MODEL, D_MODEL), :]
            w2f = w2buf[slot, :, pl.ds(f * D_MODEL, D_MODEL)]
            a = lax.dot_general(x, w1f, (((1,), (1,)), ((), ())),
                                preferred_element_type=jnp.float32)
            bb = lax.dot_general(x, v1f, (((1,), (1,)), ((), ())),
                                 preferred_element_type=jnp.float32)
            h = (a * lax.logistic(a)) * bb
            y = y + lax.dot_general(h, w2f, (((1,), (1,)), ((), ())),
                                    preferred_element_type=jnp.float32)
        yg_ref[...] = y


def _combine_body(yg_hbm, pos_hbm, wr0_hbm, wr1_hbm, out_hbm,
                  p0, p1, ya0, ya1, yb0, yb1, wb0, wb1,
                  sga, sgb, so):
    wid = lax.axis_index("s") * 2 + lax.axis_index("c")
    base = wid * TW
    pltpu.sync_copy(pos_hbm.at[0, pl.ds(base, TW)], p0)
    pltpu.sync_copy(pos_hbm.at[1, pl.ds(base, TW)], p1)
    pltpu.sync_copy(wr0_hbm.at[pl.ds(base, TW)], wb0)
    pltpu.sync_copy(wr1_hbm.at[pl.ds(base, TW)], wb1)
    CH = 16
    NCH = TW // CH
    bufs = ((ya0, ya1, sga), (yb0, yb1, sgb))

    def issue(c):
        b0, b1, sg = bufs[c % 2]
        return (pltpu.async_copy(yg_hbm.at[p0.at[pl.ds(c * CH, CH)]], b0, sg),
                pltpu.async_copy(yg_hbm.at[p1.at[pl.ds(c * CH, CH)]], b1, sg))

    pend = {0: issue(0), 1: issue(1)}
    for c in range(NCH):
        u0, u1, _ = bufs[c % 2]
        g0, g1 = pend[c]
        g0.wait()
        g1.wait()

        def add_row(j, _):
            wv0 = wb0[c * CH + j, pl.ds(0, 16)]
            wv1 = wb1[c * CH + j, pl.ds(0, 16)]
            for l in range(D_MODEL // 16):
                sl = pl.ds(l * 16, 16)
                u0[j, sl] = u0[j, sl] * wv0 + u1[j, sl] * wv1
            return 0

        lax.fori_loop(0, CH, add_row, 0)
        pltpu.sync_copy(u0, out_hbm.at[pl.ds(base + c * CH, CH)])
        if c + 2 < NCH:
            pend[c + 2] = issue(c + 2)


@jax.jit
def _run(x, w_router, ws, w2s):
    pos, wr0, wr1, bmeta = pl.pallas_call(
        _route_body,
        out_shape=(
            jax.ShapeDtypeStruct((T, TOP_K), jnp.int32),
            jax.ShapeDtypeStruct((T, 128), jnp.float32),
            jax.ShapeDtypeStruct((T, 128), jnp.float32),
            jax.ShapeDtypeStruct((8, 128), jnp.int32),
        ),
    )(x, w_router)

    pos01 = pos.T

    mesh = plsc.VectorSubcoreMesh(core_axis_name="c", subcore_axis_name="s")
    xg = pl.kernel(
        _dispatch_body,
        mesh=mesh,
        out_type=jax.ShapeDtypeStruct((NPAD, D_MODEL), jnp.float32),
        scratch_types=[
            pltpu.VMEM((TW, D_MODEL), jnp.float32),
            pltpu.VMEM((TW,), jnp.int32),
            pltpu.VMEM((TW,), jnp.int32),
            pltpu.SemaphoreType.DMA,
        ],
    )(x, pos01)

    yg = pl.pallas_call(
        _gemm_body,
        grid_spec=pltpu.PrefetchScalarGridSpec(
            num_scalar_prefetch=1,
            grid=(NB,),
            in_specs=[
                pl.BlockSpec((BM, D_MODEL), lambda b, m: (b, 0)),
                pl.BlockSpec(memory_space=pl.ANY),
                pl.BlockSpec(memory_space=pl.ANY),
            ],
            out_specs=pl.BlockSpec((BM, D_MODEL), lambda b, m: (b, 0)),
            scratch_shapes=[
                pltpu.VMEM((2, 2 * D_FF, D_MODEL), jnp.float32),
                pltpu.VMEM((2, D_MODEL, D_FF), jnp.float32),
                pltpu.SemaphoreType.DMA((2,)),
                pltpu.SemaphoreType.DMA((2,)),
            ],
        ),
        out_shape=jax.ShapeDtypeStruct((NPAD, D_MODEL), jnp.float32),
        compiler_params=pltpu.CompilerParams(
            dimension_semantics=("arbitrary",)),
    )(bmeta, xg, ws, w2s)

    out = pl.kernel(
        _combine_body,
        mesh=mesh,
        out_type=jax.ShapeDtypeStruct((T, D_MODEL), jnp.float32),
        scratch_types=[
            pltpu.VMEM((TW,), jnp.int32),
            pltpu.VMEM((TW,), jnp.int32),
            pltpu.VMEM((16, D_MODEL), jnp.float32),
            pltpu.VMEM((16, D_MODEL), jnp.float32),
            pltpu.VMEM((16, D_MODEL), jnp.float32),
            pltpu.VMEM((16, D_MODEL), jnp.float32),
            pltpu.VMEM((TW, 128), jnp.float32),
            pltpu.VMEM((TW, 128), jnp.float32),
            pltpu.SemaphoreType.DMA,
            pltpu.SemaphoreType.DMA,
            pltpu.SemaphoreType.DMA,
        ],
    )(yg, pos01, wr0, wr1)
    return out


def kernel(hidden_states, w_router, ws, w2s):
    b, s, d = hidden_states.shape
    x = hidden_states.reshape(s, d)
    out = _run(x, w_router, ws, w2s)
    return out.reshape(b, s, d)
